# baseline (speedup 1.0000x reference)
import functools

import jax
import jax.numpy as jnp
from jax import lax
from jax.experimental import pallas as pl
from jax.experimental.pallas import tpu as pltpu
from jax.experimental.pallas import tpu_sc as plsc

N = 10000
E = 320000
D = 128
L = 5
G = 128
MAX_SEQ = 5
NUM_VOCAB = 5000
MAX_DEPTH = 20

NC = 2
NS = 16
NW = NC * NS

NP = 10240
CN = NP // NW
K = 64
NCHUNK = 160
NBUF = 5
EP = NW * NCHUNK * K
ROWS_PT = NP // NS

VP = 5120
VB = 512
BN = 512

_mesh = plsc.VectorSubcoreMesh(core_axis_name="c", subcore_axis_name="s")


@functools.partial(
    pl.kernel,
    out_type=jax.ShapeDtypeStruct((NP, D), jnp.float32),
    mesh=_mesh,
    scratch_types=[
        pltpu.VMEM((CN,), jnp.int32),
        pltpu.VMEM((CN,), jnp.int32),
        pltpu.VMEM((CN,), jnp.int32),
        pltpu.VMEM((CN, D), jnp.float32),
        pltpu.VMEM((CN, D), jnp.float32),
        pltpu.SemaphoreType.DMA,
    ],
)
def _encoder(x0_hbm, x1_hbm, dep_hbm, temb_hbm, aemb_hbm, demb_hbm, out_hbm,
             i0, i1, i2, acc, buf, sem):
    wid = lax.axis_index("s") * NC + lax.axis_index("c")
    base = wid * CN
    pltpu.sync_copy(x0_hbm.at[pl.ds(base, CN)], i0)
    pltpu.sync_copy(x1_hbm.at[pl.ds(base, CN)], i1)
    pltpu.sync_copy(dep_hbm.at[pl.ds(base, CN)], i2)

    def _clip(i, c):
        sl = pl.ds(i * 16, 16)
        i2[sl] = jnp.minimum(i2[sl], MAX_DEPTH)
        return c
    lax.fori_loop(0, CN // 16, _clip, 0)

    pltpu.async_copy(temb_hbm.at[i0], acc, sem).wait()
    pltpu.async_copy(aemb_hbm.at[i1], buf, sem).wait()

    def _add(r, c):
        for s in range(8):
            sl = pl.ds(s * 16, 16)
            acc[r, sl] = acc[r, sl] + buf[r, sl]
        return c
    lax.fori_loop(0, CN, _add, 0)
    pltpu.async_copy(demb_hbm.at[i2], buf, sem).wait()
    lax.fori_loop(0, CN, _add, 0)
    pltpu.sync_copy(acc, out_hbm.at[pl.ds(base, CN)])


@functools.partial(
    pl.kernel,
    out_type=jax.ShapeDtypeStruct((NC, NP, D), jnp.float32),
    mesh=_mesh,
    scratch_types=[
        pltpu.VMEM_SHARED((NP, D), jnp.float32),
        *[pltpu.VMEM((4, K), jnp.float32) for _ in range(NBUF)],
        *[pltpu.VMEM((2, K), jnp.int32) for _ in range(NBUF)],
        *[pltpu.VMEM((K, D), jnp.float32) for _ in range(NBUF)],
        pltpu.VMEM((3, D), jnp.float32),
        *[pltpu.SemaphoreType.DMA for _ in range(3 * NBUF)],
    ],
)
def _msgpass(h_hbm, ec_hbm, w_hbm, out_hbm, acc,
             f0b, f1b, f2b, f3b, f4b, e0b, e1b, e2b, e3b, e4b,
             r0b, r1b, r2b, r3b, r4b, wv,
             i0s, i1s, i2s, i3s, i4s, g0s, g1s, g2s, g3s, g4s,
             s0s, s1s, s2s, s3s, s4s):
    fbuf = [f0b, f1b, f2b, f3b, f4b]
    ibuf = [e0b, e1b, e2b, e3b, e4b]
    rows = [r0b, r1b, r2b, r3b, r4b]
    isem = [i0s, i1s, i2s, i3s, i4s]
    gsem = [g0s, g1s, g2s, g3s, g4s]
    ssem = [s0s, s1s, s2s, s3s, s4s]

    cid = lax.axis_index("c")
    sid = lax.axis_index("s")
    wid = sid * NC + cid
    cbase = wid * NCHUNK

    pltpu.sync_copy(w_hbm, wv)

    def _z(r, c):
        for s in range(8):
            rows[0][r, pl.ds(s * 16, 16)] = jnp.zeros((16,), jnp.float32)
        return c
    lax.fori_loop(0, K, _z, 0)
    row0 = sid * ROWS_PT
    for j in range(ROWS_PT // K):
        pltpu.sync_copy(rows[0], acc.at[pl.ds(row0 + j * K, K)])
    plsc.subcore_barrier()

    def _idx_start(j, b):
        pltpu.async_copy(ec_hbm.at[cbase + j], fbuf[b], isem[b])

    def _idx_wait(b):
        pltpu.make_async_copy(ec_hbm.at[cbase], fbuf[b], isem[b]).wait()
        for r in range(2):
            for t in range(K // 16):
                sl = pl.ds(t * 16, 16)
                ibuf[b][r, sl] = fbuf[b][r, sl].astype(jnp.int32)

    def _gather_start(b):
        pltpu.async_copy(h_hbm.at[ibuf[b].at[0]], rows[b], gsem[b])

    def _gather_wait(b):
        pltpu.make_async_copy(h_hbm.at[ibuf[b].at[0]], rows[b],
                              gsem[b]).wait()

    def _scatter_start(b):
        pltpu.async_copy(rows[b], acc.at[ibuf[b].at[1]], ssem[b], add=True)

    def _scatter_wait(b):
        pltpu.make_async_copy(rows[b], acc.at[ibuf[b].at[1]],
                              ssem[b]).wait()

    def _compute(b):
        def _grp(g, c2):
            va0 = fbuf[b][2, pl.ds(g * 16, 16)]
            va1 = fbuf[b][3, pl.ds(g * 16, 16)]
            e0 = g * 16
            for i in range(16):
                ea0 = va0[i]
                ea1 = va1[i]
                for s in range(8):
                    sl = pl.ds(s * 16, 16)
                    v = rows[b][e0 + i, sl] + (
                        ea0 * wv[0, sl] + ea1 * wv[1, sl] + wv[2, sl])
                    rows[b][e0 + i, sl] = jnp.maximum(v, 0.0)
            return c2
        lax.fori_loop(0, K // 16, _grp, 0)

    _idx_start(0, 0)
    _idx_start(1, 1)
    _idx_start(2, 2)
    _idx_wait(0)
    _gather_start(0)
    _idx_wait(1)
    _gather_start(1)

    def _super(sj, c):
        for u in range(NBUF):
            j = sj * NBUF + u
            b = u

            @pl.when(j >= 2)
            def _():
                _scatter_wait((u + 3) % NBUF)

            @pl.when(j <= NCHUNK - 4)
            def _():
                _idx_start(j + 3, (u + 3) % NBUF)

            @pl.when(j <= NCHUNK - 3)
            def _():
                _idx_wait((u + 2) % NBUF)
                _gather_start((u + 2) % NBUF)

            _gather_wait(b)
            _compute(b)
            _scatter_start(b)
        return c
    lax.fori_loop(0, NCHUNK // NBUF, _super, 0)
    for j in range(NCHUNK - 2, NCHUNK):
        _scatter_wait(j % NBUF)

    plsc.subcore_barrier()
    pltpu.sync_copy(acc.at[pl.ds(row0, ROWS_PT)],
                    out_hbm.at[cid].at[pl.ds(row0, ROWS_PT)])


def _mlp_body(relu_out, href, aggref, w1ref, b1ref, w2ref, b2ref, epsref, oref):
    z = href[...] * (1.0 + epsref[0]) + aggref[0] + aggref[1]
    z1 = lax.dot_general(z, w1ref[...], (((1,), (0,)), ((), ())),
                         preferred_element_type=jnp.float32) + b1ref[...]
    z1 = jnp.maximum(z1, 0.0)
    z2 = lax.dot_general(z1, w2ref[...], (((1,), (0,)), ((), ())),
                         preferred_element_type=jnp.float32) + b2ref[...]
    oref[...] = jnp.maximum(z2, 0.0) if relu_out else z2


def _mlp(h, agg, w1, b1, w2, b2, epsl, relu_out):
    return pl.pallas_call(
        functools.partial(_mlp_body, relu_out),
        grid=(NP // BN,),
        in_specs=[
            pl.BlockSpec((BN, D), lambda i: (i, 0)),
            pl.BlockSpec((NC, BN, D), lambda i: (0, i, 0)),
            pl.BlockSpec((D, 2 * D), lambda i: (0, 0)),
            pl.BlockSpec((1, 2 * D), lambda i: (0, 0)),
            pl.BlockSpec((2 * D, D), lambda i: (0, 0)),
            pl.BlockSpec((1, D), lambda i: (0, 0)),
            pl.BlockSpec(memory_space=pltpu.SMEM),
        ],
        out_specs=pl.BlockSpec((BN, D), lambda i: (i, 0)),
        out_shape=jax.ShapeDtypeStruct((NP, D), jnp.float32),
    )(h, agg, w1, b1, w2, b2, epsl)


def _pool_body(href, bref, oref, hg_acc, cnt_acc):
    i = pl.program_id(0)

    @pl.when(i == 0)
    def _():
        hg_acc[...] = jnp.zeros_like(hg_acc)
        cnt_acc[...] = jnp.zeros_like(cnt_acc)

    iota = lax.broadcasted_iota(jnp.int32, (BN, G), 1)
    oh = (bref[...] == iota).astype(jnp.float32)
    hg_acc[...] += lax.dot_general(oh, href[...], (((0,), (0,)), ((), ())),
                                   preferred_element_type=jnp.float32)
    cnt_acc[...] += lax.dot_general(oh, jnp.ones((BN, D), jnp.float32),
                                    (((0,), (0,)), ((), ())),
                                    preferred_element_type=jnp.float32)

    @pl.when(i == NP // BN - 1)
    def _():
        oref[...] = hg_acc[...] / jnp.maximum(cnt_acc[...], 1.0)


def _pool(h, batchp):
    return pl.pallas_call(
        _pool_body,
        grid=(NP // BN,),
        in_specs=[
            pl.BlockSpec((BN, D), lambda i: (i, 0)),
            pl.BlockSpec((BN, 1), lambda i: (i, 0)),
        ],
        out_specs=pl.BlockSpec((G, D), lambda i: (0, 0)),
        out_shape=jax.ShapeDtypeStruct((G, D), jnp.float32),
        scratch_shapes=[
            pltpu.VMEM((G, D), jnp.float32),
            pltpu.VMEM((G, D), jnp.float32),
        ],
        compiler_params=pltpu.CompilerParams(
            dimension_semantics=("arbitrary",)),
    )(h, batchp)


def _heads_body(hgref, wref, bref, oref):
    oref[0] = lax.dot_general(hgref[...], wref[0], (((1,), (0,)), ((), ())),
                              preferred_element_type=jnp.float32) + bref[0]


def _heads(hg, wp, bp):
    return pl.pallas_call(
        _heads_body,
        grid=(MAX_SEQ, VP // VB),
        in_specs=[
            pl.BlockSpec((G, D), lambda i, j: (0, 0)),
            pl.BlockSpec((1, D, VB), lambda i, j: (i, 0, j)),
            pl.BlockSpec((1, 1, VB), lambda i, j: (i, 0, j)),
        ],
        out_specs=pl.BlockSpec((1, G, VB), lambda i, j: (i, 0, j)),
        out_shape=jax.ShapeDtypeStruct((MAX_SEQ, G, VP), jnp.float32),
    )(hg, wp, bp)


def kernel(x, node_depth, edge_index, edge_attr, batch, type_emb, attr_emb,
           depth_emb, edge_W, edge_b, W1, b1, bn1_g, bn1_b, W2, b2, bn2_g,
           bn2_b, eps, pred_W, pred_b):
    pad = NP - N
    x0 = jnp.pad(x[:, 0].astype(jnp.int32), (0, pad))
    x1 = jnp.pad(x[:, 1].astype(jnp.int32), (0, pad))
    dep = jnp.pad(node_depth.astype(jnp.int32), (0, pad))
    batchp = jnp.pad(batch.astype(jnp.int32), (0, pad),
                     constant_values=G).reshape(NP, 1)

    epad = EP - E
    src2 = jnp.pad(edge_index[0].astype(jnp.float32),
                   (0, epad)).reshape(EP // K, K)
    dst2 = jnp.pad(edge_index[1].astype(jnp.float32), (0, epad),
                   constant_values=float(N)).reshape(EP // K, K)
    a0 = jnp.pad(edge_attr[:, 0], (0, epad)).reshape(EP // K, K)
    a1 = jnp.pad(edge_attr[:, 1], (0, epad)).reshape(EP // K, K)
    ec = jnp.stack([src2, dst2, a0, a1], axis=1)

    inv = (1.0 + 1e-5) ** -0.5
    al1 = bn1_g * inv
    W1e = W1 * al1[:, None, :]
    b1e = b1 * al1 + bn1_b
    al2 = bn2_g * inv
    W2e = W2 * al2[:, None, :]
    b2e = b2 * al2 + bn2_b
    wmat = jnp.concatenate([edge_W, edge_b[:, None, :]], axis=1)

    h = _encoder(x0, x1, dep, type_emb, attr_emb, depth_emb)
    for l in range(L):
        agg = _msgpass(h, ec, wmat[l])
        h = _mlp(h, agg, W1e[l], b1e[l].reshape(1, -1), W2e[l],
                 b2e[l].reshape(1, -1), eps[l].reshape(1),
                 relu_out=(l != L - 1))
    hg = _pool(h, batchp)

    wp = jnp.pad(pred_W, ((0, 0), (0, 0), (0, VP - NUM_VOCAB)))
    bp = jnp.pad(pred_b, ((0, 0), (0, VP - NUM_VOCAB))).reshape(MAX_SEQ, 1, VP)
    out = _heads(hg, wp, bp)
    return tuple(out[i, :, :NUM_VOCAB] for i in range(MAX_SEQ))


# speedup vs baseline: 2.8595x; 2.8595x over previous
import functools

import jax
import jax.numpy as jnp
from jax import lax
from jax.experimental import pallas as pl
from jax.experimental.pallas import tpu as pltpu
from jax.experimental.pallas import tpu_sc as plsc

N = 10000
E = 320000
D = 128
L = 5
G = 128
MAX_SEQ = 5
NUM_VOCAB = 5000
MAX_DEPTH = 20

NC = 2
NS = 16
NW = NC * NS

NP = 10240
CN = NP // NW
K = 80
NCHUNK = 128
NBUF = 4
EP = NW * NCHUNK * K
ROWS_PT = NP // NS

VP = 5120
VB = 512
BN = 512

_mesh = plsc.VectorSubcoreMesh(core_axis_name="c", subcore_axis_name="s")


@functools.partial(
    pl.kernel,
    out_type=jax.ShapeDtypeStruct((NP, D), jnp.float32),
    mesh=_mesh,
    scratch_types=[
        pltpu.VMEM((CN,), jnp.int32),
        pltpu.VMEM((CN,), jnp.int32),
        pltpu.VMEM((CN,), jnp.int32),
        pltpu.VMEM((CN, D), jnp.float32),
        pltpu.VMEM((CN, D), jnp.float32),
        pltpu.SemaphoreType.DMA,
    ],
)
def _encoder(x0_hbm, x1_hbm, dep_hbm, temb_hbm, aemb_hbm, demb_hbm, out_hbm,
             i0, i1, i2, acc, buf, sem):
    wid = lax.axis_index("s") * NC + lax.axis_index("c")
    base = wid * CN
    pltpu.sync_copy(x0_hbm.at[pl.ds(base, CN)], i0)
    pltpu.sync_copy(x1_hbm.at[pl.ds(base, CN)], i1)
    pltpu.sync_copy(dep_hbm.at[pl.ds(base, CN)], i2)

    def _clip(i, c):
        sl = pl.ds(i * 16, 16)
        i2[sl] = jnp.minimum(i2[sl], MAX_DEPTH)
        return c
    lax.fori_loop(0, CN // 16, _clip, 0)

    pltpu.async_copy(temb_hbm.at[i0], acc, sem).wait()
    pltpu.async_copy(aemb_hbm.at[i1], buf, sem).wait()

    def _add(r, c):
        for s in range(8):
            sl = pl.ds(s * 16, 16)
            acc[r, sl] = acc[r, sl] + buf[r, sl]
        return c
    lax.fori_loop(0, CN, _add, 0)
    pltpu.async_copy(demb_hbm.at[i2], buf, sem).wait()
    lax.fori_loop(0, CN, _add, 0)
    pltpu.sync_copy(acc, out_hbm.at[pl.ds(base, CN)])


@functools.partial(
    pl.kernel,
    out_type=jax.ShapeDtypeStruct((NC, NP, D), jnp.float32),
    mesh=_mesh,
    scratch_types=[
        pltpu.VMEM_SHARED((NP, D), jnp.float32),
        *[pltpu.VMEM((4, K), jnp.float32) for _ in range(NBUF)],
        *[pltpu.VMEM((2, K), jnp.int32) for _ in range(NBUF)],
        *[pltpu.VMEM((K, D), jnp.float32) for _ in range(NBUF)],
        pltpu.VMEM((3, D), jnp.float32),
        *[pltpu.SemaphoreType.DMA for _ in range(3 * NBUF)],
    ],
)
def _msgpass(h_hbm, ec_hbm, w_hbm, out_hbm, acc,
             f0b, f1b, f2b, f3b, e0b, e1b, e2b, e3b, r0b, r1b, r2b, r3b,
             wv, i0s, i1s, i2s, i3s, g0s, g1s, g2s, g3s, s0s, s1s, s2s, s3s):
    fbuf = [f0b, f1b, f2b, f3b]
    ibuf = [e0b, e1b, e2b, e3b]
    rows = [r0b, r1b, r2b, r3b]
    isem = [i0s, i1s, i2s, i3s]
    gsem = [g0s, g1s, g2s, g3s]
    ssem = [s0s, s1s, s2s, s3s]

    cid = lax.axis_index("c")
    sid = lax.axis_index("s")
    wid = sid * NC + cid
    cbase = wid * NCHUNK

    pltpu.sync_copy(w_hbm, wv)

    def _z(r, c):
        for s in range(8):
            rows[0][r, pl.ds(s * 16, 16)] = jnp.zeros((16,), jnp.float32)
        return c
    lax.fori_loop(0, K, _z, 0)
    row0 = sid * ROWS_PT
    for j in range(ROWS_PT // K):
        pltpu.sync_copy(rows[0], acc.at[pl.ds(row0 + j * K, K)])
    plsc.subcore_barrier()

    w0 = [wv[0, pl.ds(s * 16, 16)] for s in range(8)]
    w1 = [wv[1, pl.ds(s * 16, 16)] for s in range(8)]
    wb = [wv[2, pl.ds(s * 16, 16)] for s in range(8)]

    def _idx_start(j, b):
        pltpu.async_copy(ec_hbm.at[cbase + j], fbuf[b], isem[b])

    def _idx_wait(b):
        pltpu.make_async_copy(ec_hbm.at[cbase], fbuf[b], isem[b]).wait()
        for r in range(2):
            for t in range(K // 16):
                sl = pl.ds(t * 16, 16)
                ibuf[b][r, sl] = fbuf[b][r, sl].astype(jnp.int32)

    def _gather_start(b):
        pltpu.async_copy(h_hbm.at[ibuf[b].at[0]], rows[b], gsem[b])

    def _gather_wait(b):
        pltpu.make_async_copy(h_hbm.at[ibuf[b].at[0]], rows[b],
                              gsem[b]).wait()

    def _scatter_start(b):
        pltpu.async_copy(rows[b], acc.at[ibuf[b].at[1]], ssem[b], add=True)

    def _scatter_wait(b):
        pltpu.make_async_copy(rows[b], acc.at[ibuf[b].at[1]],
                              ssem[b]).wait()

    def _compute(b):
        def _grp(g, c2):
            va0 = fbuf[b][2, pl.ds(g * 16, 16)]
            va1 = fbuf[b][3, pl.ds(g * 16, 16)]
            e0 = g * 16
            for i in range(16):
                ea0 = va0[i]
                ea1 = va1[i]
                for s in range(8):
                    sl = pl.ds(s * 16, 16)
                    v = rows[b][e0 + i, sl] + (
                        ea0 * w0[s] + ea1 * w1[s] + wb[s])
                    rows[b][e0 + i, sl] = jnp.maximum(v, 0.0)
            return c2
        lax.fori_loop(0, K // 16, _grp, 0)

    _idx_start(0, 0)
    _idx_start(1, 1)
    _idx_wait(0)
    _gather_start(0)

    def _super(sj, c):
        for u in range(NBUF):
            j = sj * NBUF + u
            b = u

            @pl.when(jnp.logical_and(j >= 2, j <= NCHUNK - 3))
            def _():
                _scatter_wait((u + 2) % NBUF)

            @pl.when(j <= NCHUNK - 3)
            def _():
                _idx_start(j + 2, (u + 2) % NBUF)

            @pl.when(j <= NCHUNK - 2)
            def _():
                _idx_wait((u + 1) % NBUF)
                _gather_start((u + 1) % NBUF)

            _gather_wait(b)
            _compute(b)
            _scatter_start(b)
        return c
    lax.fori_loop(0, NCHUNK // NBUF, _super, 0)
    for b in range(NBUF):
        _scatter_wait(b)

    plsc.subcore_barrier()
    pltpu.sync_copy(acc.at[pl.ds(row0, ROWS_PT)],
                    out_hbm.at[cid].at[pl.ds(row0, ROWS_PT)])


def _mlp_body(relu_out, href, aggref, w1ref, b1ref, w2ref, b2ref, epsref, oref):
    z = href[...] * (1.0 + epsref[0]) + aggref[0] + aggref[1]
    z1 = lax.dot_general(z, w1ref[...], (((1,), (0,)), ((), ())),
                         preferred_element_type=jnp.float32) + b1ref[...]
    z1 = jnp.maximum(z1, 0.0)
    z2 = lax.dot_general(z1, w2ref[...], (((1,), (0,)), ((), ())),
                         preferred_element_type=jnp.float32) + b2ref[...]
    oref[...] = jnp.maximum(z2, 0.0) if relu_out else z2


def _mlp(h, agg, w1, b1, w2, b2, epsl, relu_out):
    return pl.pallas_call(
        functools.partial(_mlp_body, relu_out),
        grid=(NP // BN,),
        in_specs=[
            pl.BlockSpec((BN, D), lambda i: (i, 0)),
            pl.BlockSpec((NC, BN, D), lambda i: (0, i, 0)),
            pl.BlockSpec((D, 2 * D), lambda i: (0, 0)),
            pl.BlockSpec((1, 2 * D), lambda i: (0, 0)),
            pl.BlockSpec((2 * D, D), lambda i: (0, 0)),
            pl.BlockSpec((1, D), lambda i: (0, 0)),
            pl.BlockSpec(memory_space=pltpu.SMEM),
        ],
        out_specs=pl.BlockSpec((BN, D), lambda i: (i, 0)),
        out_shape=jax.ShapeDtypeStruct((NP, D), jnp.float32),
    )(h, agg, w1, b1, w2, b2, epsl)


def _pool_body(href, bref, oref, hg_acc, cnt_acc):
    i = pl.program_id(0)

    @pl.when(i == 0)
    def _():
        hg_acc[...] = jnp.zeros_like(hg_acc)
        cnt_acc[...] = jnp.zeros_like(cnt_acc)

    iota = lax.broadcasted_iota(jnp.int32, (BN, G), 1)
    oh = (bref[...] == iota).astype(jnp.float32)
    hg_acc[...] += lax.dot_general(oh, href[...], (((0,), (0,)), ((), ())),
                                   preferred_element_type=jnp.float32)
    cnt_acc[...] += lax.dot_general(oh, jnp.ones((BN, D), jnp.float32),
                                    (((0,), (0,)), ((), ())),
                                    preferred_element_type=jnp.float32)

    @pl.when(i == NP // BN - 1)
    def _():
        oref[...] = hg_acc[...] / jnp.maximum(cnt_acc[...], 1.0)


def _pool(h, batchp):
    return pl.pallas_call(
        _pool_body,
        grid=(NP // BN,),
        in_specs=[
            pl.BlockSpec((BN, D), lambda i: (i, 0)),
            pl.BlockSpec((BN, 1), lambda i: (i, 0)),
        ],
        out_specs=pl.BlockSpec((G, D), lambda i: (0, 0)),
        out_shape=jax.ShapeDtypeStruct((G, D), jnp.float32),
        scratch_shapes=[
            pltpu.VMEM((G, D), jnp.float32),
            pltpu.VMEM((G, D), jnp.float32),
        ],
        compiler_params=pltpu.CompilerParams(
            dimension_semantics=("arbitrary",)),
    )(h, batchp)


def _heads_body(hgref, wref, bref, oref):
    oref[0] = lax.dot_general(hgref[...], wref[0], (((1,), (0,)), ((), ())),
                              preferred_element_type=jnp.float32) + bref[0]


def _heads(hg, wp, bp):
    return pl.pallas_call(
        _heads_body,
        grid=(MAX_SEQ, VP // VB),
        in_specs=[
            pl.BlockSpec((G, D), lambda i, j: (0, 0)),
            pl.BlockSpec((1, D, VB), lambda i, j: (i, 0, j)),
            pl.BlockSpec((1, 1, VB), lambda i, j: (i, 0, j)),
        ],
        out_specs=pl.BlockSpec((1, G, VB), lambda i, j: (i, 0, j)),
        out_shape=jax.ShapeDtypeStruct((MAX_SEQ, G, VP), jnp.float32),
    )(hg, wp, bp)


def kernel(x, node_depth, edge_index, edge_attr, batch, type_emb, attr_emb,
           depth_emb, edge_W, edge_b, W1, b1, bn1_g, bn1_b, W2, b2, bn2_g,
           bn2_b, eps, pred_W, pred_b):
    pad = NP - N
    x0 = jnp.pad(x[:, 0].astype(jnp.int32), (0, pad))
    x1 = jnp.pad(x[:, 1].astype(jnp.int32), (0, pad))
    dep = jnp.pad(node_depth.astype(jnp.int32), (0, pad))
    batchp = jnp.pad(batch.astype(jnp.int32), (0, pad),
                     constant_values=G).reshape(NP, 1)

    epad = EP - E
    src2 = jnp.pad(edge_index[0].astype(jnp.float32),
                   (0, epad)).reshape(EP // K, K)
    dst2 = jnp.pad(edge_index[1].astype(jnp.float32), (0, epad),
                   constant_values=float(N)).reshape(EP // K, K)
    a0 = jnp.pad(edge_attr[:, 0], (0, epad)).reshape(EP // K, K)
    a1 = jnp.pad(edge_attr[:, 1], (0, epad)).reshape(EP // K, K)
    ec = jnp.stack([src2, dst2, a0, a1], axis=1)

    inv = (1.0 + 1e-5) ** -0.5
    al1 = bn1_g * inv
    W1e = W1 * al1[:, None, :]
    b1e = b1 * al1 + bn1_b
    al2 = bn2_g * inv
    W2e = W2 * al2[:, None, :]
    b2e = b2 * al2 + bn2_b
    wmat = jnp.concatenate([edge_W, edge_b[:, None, :]], axis=1)

    h = _encoder(x0, x1, dep, type_emb, attr_emb, depth_emb)
    for l in range(L):
        agg = _msgpass(h, ec, wmat[l])
        h = _mlp(h, agg, W1e[l], b1e[l].reshape(1, -1), W2e[l],
                 b2e[l].reshape(1, -1), eps[l].reshape(1),
                 relu_out=(l != L - 1))
    hg = _pool(h, batchp)

    wp = jnp.pad(pred_W, ((0, 0), (0, 0), (0, VP - NUM_VOCAB)))
    bp = jnp.pad(pred_b, ((0, 0), (0, VP - NUM_VOCAB))).reshape(MAX_SEQ, 1, VP)
    out = _heads(hg, wp, bp)
    return tuple(out[i, :, :NUM_VOCAB] for i in range(MAX_SEQ))
